# baseline (speedup 1.0000x reference)
import jax
import jax.numpy as jnp
from jax.experimental import pallas as pl
from jax.experimental.pallas import tpu as pltpu

_NUM_ANCHORS = 3
_NUM_CLASSES = 5
_ATTRS = 5 + _NUM_CLASSES
_STRIDE = 4.0


def _decode_kernel(in_ref, out_ref):
    a = pl.program_id(0) % _NUM_ANCHORS
    d = pl.program_id(1)
    v = in_ref[0, :, 0]

    aw = jnp.where(a == 0, 10.0, jnp.where(a == 1, 16.0, 33.0)).astype(jnp.float32)

    gx = jax.lax.broadcasted_iota(jnp.int32, (1, 64, 64), 2).astype(jnp.float32)
    gy = jax.lax.broadcasted_iota(jnp.int32, (1, 64, 64), 1).astype(jnp.float32)
    gz = jnp.full((1, 64, 64), d.astype(jnp.float32), dtype=jnp.float32)
    grid_off = jnp.concatenate([gx, gy, gz], axis=0)

    bxyz = (jax.nn.sigmoid(v[0:3]) + grid_off) * _STRIDE
    bl = jnp.exp(v[3:4]) * aw
    rest = jax.nn.sigmoid(v[4:10])
    res = jnp.concatenate([bxyz, bl, rest], axis=0)

    out_ref[0, 0, 0] = jnp.transpose(res, (1, 2, 0)).reshape(64 * 64, _ATTRS)


def kernel(input):
    B = input.shape[0]
    D, H, W = input.shape[2], input.shape[3], input.shape[4]
    hw = H * W

    out = pl.pallas_call(
        _decode_kernel,
        grid=(B * _NUM_ANCHORS, D),
        in_specs=[
            pl.BlockSpec(
                (1, _ATTRS, 1, H, W),
                lambda i, j: (i // _NUM_ANCHORS, i % _NUM_ANCHORS, j, 0, 0),
            )
        ],
        out_specs=pl.BlockSpec(
            (1, 1, 1, hw, _ATTRS),
            lambda i, j: (i // _NUM_ANCHORS, i % _NUM_ANCHORS, j, 0, 0),
        ),
        out_shape=jax.ShapeDtypeStruct((B, _NUM_ANCHORS, D, hw, _ATTRS), jnp.float32),
        compiler_params=pltpu.CompilerParams(
            dimension_semantics=("parallel", "arbitrary"),
        ),
    )(input)
    return out.reshape(B, _NUM_ANCHORS * D * hw, _ATTRS)


# speedup vs baseline: 3.3999x; 3.3999x over previous
import jax
import jax.numpy as jnp
from jax.experimental import pallas as pl
from jax.experimental.pallas import tpu as pltpu

_NUM_ANCHORS = 3
_NUM_CLASSES = 5
_ATTRS = 5 + _NUM_CLASSES
_STRIDE = 4.0
_ANCHOR_W = (10.0, 16.0, 33.0)
_DBLK = 8


def _to_lanes128(x):
    d = x.shape[0]
    xm = x.reshape(d * 64, 64)
    x3 = xm.reshape(d * 32, 2, 64)
    return jnp.concatenate([x3[:, 0, :], x3[:, 1, :]], axis=-1)


def _decode_kernel(in_ref, out_ref):
    a = pl.program_id(1)
    dj = pl.program_id(0)
    aw = jnp.where(a == 0, _ANCHOR_W[0],
                   jnp.where(a == 1, _ANCHOR_W[1], _ANCHOR_W[2]))
    aw = aw.astype(jnp.float32)

    shape = (_DBLK, 64, 64)
    gx = jax.lax.broadcasted_iota(jnp.int32, shape, 2).astype(jnp.float32)
    gy = jax.lax.broadcasted_iota(jnp.int32, shape, 1).astype(jnp.float32)
    gz = (jax.lax.broadcasted_iota(jnp.int32, shape, 0)
          + dj * _DBLK).astype(jnp.float32)
    grids = (gx, gy, gz)

    for b in range(4):
        for c in range(_ATTRS):
            v = in_ref[b, c]
            if c < 3:
                r = (jax.nn.sigmoid(v) + grids[c]) * _STRIDE
            elif c == 3:
                r = jnp.exp(v) * aw
            else:
                r = jax.nn.sigmoid(v)
            out_ref[c, b, 0] = _to_lanes128(r)


def kernel(input):
    B = input.shape[0]
    D, H, W = input.shape[2], input.shape[3], input.shape[4]
    hw = H * W
    n = _NUM_ANCHORS * D * hw

    out = pl.pallas_call(
        _decode_kernel,
        grid=(D // _DBLK, _NUM_ANCHORS),
        in_specs=[
            pl.BlockSpec(
                (B, _ATTRS, _DBLK, H, W),
                lambda dj, a: (0, a, dj, 0, 0),
            )
        ],
        out_specs=pl.BlockSpec(
            (_ATTRS, B, 1, _DBLK * (hw // 128), 128),
            lambda dj, a: (0, 0, a, dj, 0),
        ),
        out_shape=jax.ShapeDtypeStruct(
            (_ATTRS, B, _NUM_ANCHORS, D * (hw // 128), 128), jnp.float32
        ),
        compiler_params=pltpu.CompilerParams(
            dimension_semantics=("parallel", "arbitrary"),
        ),
    )(input)
    return out.reshape(_ATTRS, B, n).transpose(1, 2, 0)


# speedup vs baseline: 3.7628x; 1.1067x over previous
import jax
import jax.numpy as jnp
from jax.experimental import pallas as pl
from jax.experimental.pallas import tpu as pltpu

_NUM_ANCHORS = 3
_NUM_CLASSES = 5
_ATTRS = 5 + _NUM_CLASSES
_STRIDE = 4.0
_ANCHOR_W = (10.0, 16.0, 33.0)
_DBLK = 8


def _to_lanes128(x):
    d = x.shape[0]
    xm = x.reshape(d * 64, 64)
    x3 = xm.reshape(d * 32, 2, 64)
    y = jnp.concatenate([x3[:, 0, :], x3[:, 1, :]], axis=-1)
    return y.reshape(d, 32, 1, 128)


def _decode_kernel(in_ref, out_ref):
    a = pl.program_id(1)
    dj = pl.program_id(0)
    aw = jnp.where(a == 0, _ANCHOR_W[0],
                   jnp.where(a == 1, _ANCHOR_W[1], _ANCHOR_W[2]))
    aw = aw.astype(jnp.float32)

    shape = (_DBLK, 64, 64)
    gx = jax.lax.broadcasted_iota(jnp.int32, shape, 2).astype(jnp.float32)
    gy = jax.lax.broadcasted_iota(jnp.int32, shape, 1).astype(jnp.float32)
    gz = (jax.lax.broadcasted_iota(jnp.int32, shape, 0)
          + dj * _DBLK).astype(jnp.float32)
    grids = (gx, gy, gz)

    for c in range(_ATTRS):
        ys = []
        for b in range(4):
            v = in_ref[b, c]
            if c < 3:
                r = (jax.nn.sigmoid(v) + grids[c]) * _STRIDE
            elif c == 3:
                r = jnp.exp(v) * aw
            else:
                r = jax.nn.sigmoid(v)
            ys.append(_to_lanes128(r))
        out_ref[c, 0] = jnp.concatenate(ys, axis=2)


def kernel(input):
    B = input.shape[0]
    D, H, W = input.shape[2], input.shape[3], input.shape[4]
    hw = H * W
    j = hw // 128

    out = pl.pallas_call(
        _decode_kernel,
        grid=(D // _DBLK, _NUM_ANCHORS),
        in_specs=[
            pl.BlockSpec(
                (B, _ATTRS, _DBLK, H, W),
                lambda dj, a: (0, a, dj, 0, 0),
            )
        ],
        out_specs=pl.BlockSpec(
            (_ATTRS, 1, _DBLK, j, B, 128),
            lambda dj, a: (0, a, dj, 0, 0, 0),
        ),
        out_shape=jax.ShapeDtypeStruct(
            (_ATTRS, _NUM_ANCHORS, D, j, B, 128), jnp.float32
        ),
        compiler_params=pltpu.CompilerParams(
            dimension_semantics=("parallel", "arbitrary"),
        ),
    )(input)
    return out.transpose(4, 1, 2, 3, 5, 0).reshape(B, _NUM_ANCHORS * D * hw, _ATTRS)


# speedup vs baseline: 10.3842x; 2.7597x over previous
import jax
import jax.numpy as jnp
from jax.experimental import pallas as pl
from jax.experimental.pallas import tpu as pltpu

_NUM_ANCHORS = 3
_NUM_CLASSES = 5
_ATTRS = 5 + _NUM_CLASSES
_STRIDE = 4.0
_ANCHOR_W = (10.0, 16.0, 33.0)
_DBLK = 8


def _decode_kernel(in_ref, out_ref):
    a = pl.program_id(1)
    dj = pl.program_id(0)
    aw = jnp.where(a == 0, _ANCHOR_W[0],
                   jnp.where(a == 1, _ANCHOR_W[1], _ANCHOR_W[2]))
    aw = aw.astype(jnp.float32)

    shape = (_DBLK, 32, 128)
    lane = jax.lax.broadcasted_iota(jnp.int32, shape, 2)
    gx = (lane % 64).astype(jnp.float32)
    gy = (2 * jax.lax.broadcasted_iota(jnp.int32, shape, 1)
          + (lane // 64)).astype(jnp.float32)
    gz = (jax.lax.broadcasted_iota(jnp.int32, shape, 0)
          + dj * _DBLK).astype(jnp.float32)
    grids = (gx, gy, gz)

    for c in range(_ATTRS):
        for b in range(4):
            xe = in_ref[b, c, :, pl.Slice(0, 32, 2), :]
            xo = in_ref[b, c, :, pl.Slice(1, 32, 2), :]
            v = jnp.concatenate([xe, xo], axis=-1)
            if c < 3:
                r = (jax.nn.sigmoid(v) + grids[c]) * _STRIDE
            elif c == 3:
                r = jnp.exp(v) * aw
            else:
                r = jax.nn.sigmoid(v)
            out_ref[c, 0, pl.Slice(b, _DBLK * 32, 4), :] = r.reshape(
                _DBLK * 32, 128)


def kernel(input):
    B = input.shape[0]
    D, H, W = input.shape[2], input.shape[3], input.shape[4]
    hw = H * W
    n = _NUM_ANCHORS * D * hw

    out = pl.pallas_call(
        _decode_kernel,
        grid=(D // _DBLK, _NUM_ANCHORS),
        in_specs=[
            pl.BlockSpec(
                (B, _ATTRS, _DBLK, H, W),
                lambda dj, a: (0, a, dj, 0, 0),
            )
        ],
        out_specs=pl.BlockSpec(
            (_ATTRS, 1, _DBLK * (hw // 128) * B, 128),
            lambda dj, a: (0, a, dj, 0),
        ),
        out_shape=jax.ShapeDtypeStruct(
            (_ATTRS, _NUM_ANCHORS, D * (hw // 128) * B, 128), jnp.float32
        ),
        compiler_params=pltpu.CompilerParams(
            dimension_semantics=("parallel", "arbitrary"),
        ),
    )(input)
    out = out.reshape(_ATTRS, _NUM_ANCHORS, D * (hw // 128), B, 128)
    return out.transpose(3, 1, 2, 4, 0).reshape(B, n, _ATTRS)
